# Initial kernel scaffold; baseline (speedup 1.0000x reference)
import jax
import jax.numpy as jnp
from jax.experimental import pallas as pl


def kernel(pos, edge_index, batch):
    raise NotImplementedError("write your pallas kernel here")


# speedup vs baseline: 16.3520x; 16.3520x over previous
import functools

import jax
import jax.numpy as jnp
from jax import lax
from jax.experimental import pallas as pl
from jax.experimental.pallas import tpu as pltpu
from jax.experimental.pallas import tpu_sc as plsc

N_NODES = 100000
N_EDGES = 6400000
N_GRAPHS = 8

_INFO = plsc.get_sparse_core_info()
NC = _INFO.num_cores
NS = _INFO.num_subcores
NW = NC * NS
EPT = N_EDGES // NW
CE = 2000
NCHUNKS = EPT // CE
SUB = 80
NSUB = CE // SUB
NE_SUB = N_EDGES // SUB
BSC = 20000
NBCHUNKS = N_NODES // BSC

_mesh = plsc.VectorSubcoreMesh(core_axis_name="c", subcore_axis_name="s")


@functools.partial(
    pl.kernel,
    mesh=_mesh,
    compiler_params=pltpu.CompilerParams(
        needs_layout_passes=False, use_tc_tiling_on_sc=False),
    out_type=(
        jax.ShapeDtypeStruct((N_EDGES * 4,), jnp.float32),
        jax.ShapeDtypeStruct((NW, 16), jnp.int32),
    ),
    scratch_types=[
        pltpu.VMEM((NSUB, SUB), jnp.int32),
        pltpu.VMEM((NSUB, SUB), jnp.int32),
        pltpu.VMEM((CE, 8), jnp.float32),
        pltpu.VMEM((CE, 8), jnp.float32),
        pltpu.VMEM((CE * 4,), jnp.float32),
        pltpu.VMEM((BSC,), jnp.int32),
        pltpu.VMEM((16,), jnp.int32),
        pltpu.SemaphoreType.DMA,
    ],
)
def _sc_body(pos8, e2, batch, out, partials,
             jv2, iv2, rowsj, rowsi, outb, bbuf, nbv, sem):
    cid = lax.axis_index("c")
    sid = lax.axis_index("s")
    wid = sid * NC + cid
    lane = lax.iota(jnp.int32, 16)

    def dyn_gather(v, idx):
        return lax.gather(
            v, idx[:, None],
            dimension_numbers=lax.GatherDimensionNumbers(
                offset_dims=(), collapsed_slice_dims=(0,),
                start_index_map=(0,)),
            slice_sizes=(1,),
            mode=lax.GatherScatterMode.PROMISE_IN_BOUNDS)

    def lane_sum(v):
        for sh in (8, 4, 2, 1):
            v = v + dyn_gather(v, lane ^ sh)
        return v

    zero16 = jnp.zeros((16,), jnp.int32)
    c0 = jnp.full((16,), 0, jnp.int32)
    c1 = jnp.full((16,), 1, jnp.int32)
    c2 = jnp.full((16,), 2, jnp.int32)

    def bscan(b, accs):
        pltpu.sync_copy(batch.at[pl.ds(b * BSC, BSC)], bbuf)

        def binner(t, accs):
            bv = bbuf[pl.ds(t * 16, 16)]
            return tuple(
                a + jnp.where(bv >= (g + 1), 1, 0) for g, a in enumerate(accs)
            )

        return lax.fori_loop(0, BSC // 16, binner, accs)

    accs = lax.fori_loop(0, NBCHUNKS, bscan, (zero16,) * 7)
    bounds = [N_NODES - lane_sum(a) for a in accs]

    def chunk(k, cnts):
        base = wid * EPT + k * CE
        rowb = base // SUB
        pltpu.sync_copy(e2.at[pl.ds(rowb, NSUB)], jv2)
        pltpu.sync_copy(e2.at[pl.ds(NE_SUB + rowb, NSUB)], iv2)

        def gsub(s, carry):
            o = s * SUB
            cpj = pltpu.async_copy(
                pos8.at[jv2.at[s]], rowsj.at[pl.ds(o, SUB)], sem)
            cpi = pltpu.async_copy(
                pos8.at[iv2.at[s]], rowsi.at[pl.ds(o, SUB)], sem)
            cpj.wait()
            cpi.wait()
            return carry
        lax.fori_loop(0, NSUB, gsub, 0)

        def einner(t, cnts):
            ev = t * 16 + lane
            xj = plsc.load_gather(rowsj, [ev, c0])
            yj = plsc.load_gather(rowsj, [ev, c1])
            zj = plsc.load_gather(rowsj, [ev, c2])
            xi = plsc.load_gather(rowsi, [ev, c0])
            yi = plsc.load_gather(rowsi, [ev, c1])
            zi = plsc.load_gather(rowsi, [ev, c2])
            dx = xj - xi
            dy = yj - yi
            dz = zj - zi
            ss = dx * dx + dy * dy + dz * dz + 1e-12
            y = plsc.bitcast(
                0x5F3759DF - (plsc.bitcast(ss, jnp.int32) >> 1), jnp.float32
            )
            h = ss * 0.5
            y = y * (1.5 - h * y * y)
            y = y * (1.5 - h * y * y)
            y = y * (1.5 - h * y * y)
            nrm = ss * y
            ev4 = ev * 4
            plsc.store_scatter(outb, [ev4], dx)
            plsc.store_scatter(outb, [ev4 + 1], dy)
            plsc.store_scatter(outb, [ev4 + 2], dz)
            plsc.store_scatter(outb, [ev4 + 3], nrm)
            irow = jnp.full((16,), t // 5, jnp.int32)
            icol = (t % 5) * 16 + lane
            ivals = plsc.load_gather(iv2, [irow, icol])
            return tuple(
                cn + jnp.where(ivals >= bounds[g], 1, 0)
                for g, cn in enumerate(cnts)
            )

        cnts = lax.fori_loop(0, CE // 16, einner, cnts)
        pltpu.sync_copy(outb, out.at[pl.ds(base * 4, CE * 4)])
        return cnts

    cnts = lax.fori_loop(0, NCHUNKS, chunk, (zero16,) * 7)

    cge = ([jnp.full((16,), EPT, jnp.int32)]
           + [lane_sum(cn) for cn in cnts]
           + [zero16])
    nb = jnp.zeros((16,), jnp.int32)
    for g in range(N_GRAPHS):
        nb = nb + jnp.where(lane == g, cge[g] - cge[g + 1], 0)
    nbv[...] = nb
    pltpu.sync_copy(nbv, partials.at[wid])


def kernel(pos, edge_index, batch):
    pos8 = jnp.pad(pos, ((0, 0), (0, 5)))
    e2 = edge_index.reshape(2 * NE_SUB, SUB)
    out, partials = _sc_body(pos8, e2, batch)
    out = out.reshape(N_EDGES, 4)
    neighbors = jnp.sum(partials, axis=0)[:N_GRAPHS].astype(jnp.int32)
    zeros = jnp.zeros((N_EDGES, 3), pos.dtype)
    return (out, zeros, zeros, neighbors)


# speedup vs baseline: 20.5089x; 1.2542x over previous
import functools

import jax
import jax.numpy as jnp
from jax import lax
from jax.experimental import pallas as pl
from jax.experimental.pallas import tpu as pltpu
from jax.experimental.pallas import tpu_sc as plsc

N_NODES = 100000
N_EDGES = 6400000
N_GRAPHS = 8

_INFO = plsc.get_sparse_core_info()
NC = _INFO.num_cores
NS = _INFO.num_subcores
NW = NC * NS
EPT = N_EDGES // NW
CE = 2000
NCHUNKS = EPT // CE
SUB = 80
NSUB = CE // SUB
NE_SUB = N_EDGES // SUB
BSC = 20000
NBCHUNKS = N_NODES // BSC

_mesh = plsc.VectorSubcoreMesh(core_axis_name="c", subcore_axis_name="s")


@functools.partial(
    pl.kernel,
    mesh=_mesh,
    compiler_params=pltpu.CompilerParams(
        needs_layout_passes=False, use_tc_tiling_on_sc=False),
    out_type=(
        jax.ShapeDtypeStruct((N_EDGES * 4,), jnp.float32),
        jax.ShapeDtypeStruct((NW, 16), jnp.int32),
    ),
    scratch_types=[
        pltpu.VMEM((NSUB, SUB), jnp.int32),
        pltpu.VMEM((NSUB, SUB), jnp.int32),
        pltpu.VMEM((CE, 8), jnp.float32),
        pltpu.VMEM((CE, 8), jnp.float32),
        pltpu.VMEM((CE * 4,), jnp.float32),
        pltpu.VMEM((BSC,), jnp.int32),
        pltpu.VMEM((16,), jnp.int32),
        pltpu.SemaphoreType.DMA,
    ],
)
def _sc_body(pos8, e2, batch, out, partials,
             jv2, iv2, rowsj, rowsi, outb, bbuf, nbv, sem):
    cid = lax.axis_index("c")
    sid = lax.axis_index("s")
    wid = sid * NC + cid
    lane = lax.iota(jnp.int32, 16)

    def dyn_gather(v, idx):
        return lax.gather(
            v, idx[:, None],
            dimension_numbers=lax.GatherDimensionNumbers(
                offset_dims=(), collapsed_slice_dims=(0,),
                start_index_map=(0,)),
            slice_sizes=(1,),
            mode=lax.GatherScatterMode.PROMISE_IN_BOUNDS)

    def lane_sum(v):
        for sh in (8, 4, 2, 1):
            v = v + dyn_gather(v, lane ^ sh)
        return v

    zero16 = jnp.zeros((16,), jnp.int32)
    c0 = jnp.full((16,), 0, jnp.int32)
    c1 = jnp.full((16,), 1, jnp.int32)
    c2 = jnp.full((16,), 2, jnp.int32)

    def bscan(b, accs):
        pltpu.sync_copy(batch.at[pl.ds(b * BSC, BSC)], bbuf)

        def binner(t, accs):
            bv = bbuf[pl.ds(t * 16, 16)]
            return tuple(
                a + jnp.where(bv >= (g + 1), 1, 0) for g, a in enumerate(accs)
            )

        return lax.fori_loop(0, BSC // 16, binner, accs)

    accs = lax.fori_loop(0, NBCHUNKS, bscan, (zero16,) * 7)
    bounds = [N_NODES - lane_sum(a) for a in accs]

    def chunk(k, cnts):
        base = wid * EPT + k * CE
        rowb = base // SUB
        pltpu.sync_copy(e2.at[pl.ds(rowb, NSUB)], jv2)
        pltpu.sync_copy(e2.at[pl.ds(NE_SUB + rowb, NSUB)], iv2)

        def gsub(s, carry):
            o = s * SUB
            pltpu.async_copy(
                pos8.at[jv2.at[s]], rowsj.at[pl.ds(o, SUB)], sem)
            pltpu.async_copy(
                pos8.at[iv2.at[s]], rowsi.at[pl.ds(o, SUB)], sem)
            return carry
        lax.fori_loop(0, NSUB, gsub, 0)
        pltpu.make_async_copy(pos8.at[pl.ds(0, CE)], rowsj, sem).wait()
        pltpu.make_async_copy(pos8.at[pl.ds(0, CE)], rowsi, sem).wait()

        def einner(t, cnts):
            ev = t * 16 + lane
            xj = plsc.load_gather(rowsj, [ev, c0])
            yj = plsc.load_gather(rowsj, [ev, c1])
            zj = plsc.load_gather(rowsj, [ev, c2])
            xi = plsc.load_gather(rowsi, [ev, c0])
            yi = plsc.load_gather(rowsi, [ev, c1])
            zi = plsc.load_gather(rowsi, [ev, c2])
            dx = xj - xi
            dy = yj - yi
            dz = zj - zi
            ss = dx * dx + dy * dy + dz * dz + 1e-12
            y = plsc.bitcast(
                0x5F3759DF - (plsc.bitcast(ss, jnp.int32) >> 1), jnp.float32
            )
            h = ss * 0.5
            y = y * (1.5 - h * y * y)
            y = y * (1.5 - h * y * y)
            y = y * (1.5 - h * y * y)
            nrm = ss * y
            ev4 = ev * 4
            plsc.store_scatter(outb, [ev4], dx)
            plsc.store_scatter(outb, [ev4 + 1], dy)
            plsc.store_scatter(outb, [ev4 + 2], dz)
            plsc.store_scatter(outb, [ev4 + 3], nrm)
            irow = jnp.full((16,), t // 5, jnp.int32)
            icol = (t % 5) * 16 + lane
            ivals = plsc.load_gather(iv2, [irow, icol])
            return tuple(
                cn + jnp.where(ivals >= bounds[g], 1, 0)
                for g, cn in enumerate(cnts)
            )

        cnts = lax.fori_loop(0, CE // 16, einner, cnts)
        pltpu.sync_copy(outb, out.at[pl.ds(base * 4, CE * 4)])
        return cnts

    cnts = lax.fori_loop(0, NCHUNKS, chunk, (zero16,) * 7)

    cge = ([jnp.full((16,), EPT, jnp.int32)]
           + [lane_sum(cn) for cn in cnts]
           + [zero16])
    nb = jnp.zeros((16,), jnp.int32)
    for g in range(N_GRAPHS):
        nb = nb + jnp.where(lane == g, cge[g] - cge[g + 1], 0)
    nbv[...] = nb
    pltpu.sync_copy(nbv, partials.at[wid])


def kernel(pos, edge_index, batch):
    pos8 = jnp.pad(pos, ((0, 0), (0, 5)))
    e2 = edge_index.reshape(2 * NE_SUB, SUB)
    out, partials = _sc_body(pos8, e2, batch)
    out = out.reshape(N_EDGES, 4)
    neighbors = jnp.sum(partials, axis=0)[:N_GRAPHS].astype(jnp.int32)
    zeros = jnp.zeros((N_EDGES, 3), pos.dtype)
    return (out, zeros, zeros, neighbors)


# speedup vs baseline: 26.7673x; 1.3052x over previous
import functools

import jax
import jax.numpy as jnp
from jax import lax
from jax.experimental import pallas as pl
from jax.experimental.pallas import tpu as pltpu
from jax.experimental.pallas import tpu_sc as plsc

N_NODES = 100000
N_EDGES = 6400000
N_GRAPHS = 8

_INFO = plsc.get_sparse_core_info()
NC = _INFO.num_cores
NS = _INFO.num_subcores
NW = NC * NS
EPT = N_EDGES // NW
CE = 2000
NCHUNKS = EPT // CE
SUB = 80
NSUB = CE // SUB
NE_SUB = N_EDGES // SUB
BSC = 20000
NBCHUNKS = N_NODES // BSC

_mesh = plsc.VectorSubcoreMesh(core_axis_name="c", subcore_axis_name="s")


@functools.partial(
    pl.kernel,
    mesh=_mesh,
    compiler_params=pltpu.CompilerParams(
        needs_layout_passes=False, use_tc_tiling_on_sc=False),
    out_type=(
        jax.ShapeDtypeStruct((N_EDGES, 4), jnp.float32),
        jax.ShapeDtypeStruct((NW, 16), jnp.int32),
    ),
    scratch_types=[
        pltpu.VMEM((CE,), jnp.int32),
        pltpu.VMEM((CE,), jnp.int32),
        pltpu.VMEM((CE, 8), jnp.float32),
        pltpu.VMEM((CE, 8), jnp.float32),
        pltpu.VMEM((CE, 4), jnp.float32),
        pltpu.VMEM((BSC,), jnp.int32),
        pltpu.VMEM((16,), jnp.int32),
        pltpu.SemaphoreType.DMA,
    ],
)
def _sc_body(pos8, eflat, batch, out, partials,
             jv, iv, rowsj, rowsi, outb, bbuf, nbv, sem):
    cid = lax.axis_index("c")
    sid = lax.axis_index("s")
    wid = sid * NC + cid
    lane = lax.iota(jnp.int32, 16)

    def dyn_gather(v, idx):
        return lax.gather(
            v, idx[:, None],
            dimension_numbers=lax.GatherDimensionNumbers(
                offset_dims=(), collapsed_slice_dims=(0,),
                start_index_map=(0,)),
            slice_sizes=(1,),
            mode=lax.GatherScatterMode.PROMISE_IN_BOUNDS)

    def lane_sum(v):
        for sh in (8, 4, 2, 1):
            v = v + dyn_gather(v, lane ^ sh)
        return v

    zero16 = jnp.zeros((16,), jnp.int32)
    c0 = jnp.full((16,), 0, jnp.int32)
    c1 = jnp.full((16,), 1, jnp.int32)
    c2 = jnp.full((16,), 2, jnp.int32)
    c3 = jnp.full((16,), 3, jnp.int32)

    def bscan(b, accs):
        pltpu.sync_copy(batch.at[pl.ds(b * BSC, BSC)], bbuf)

        def binner(t, accs):
            bv = bbuf[pl.ds(t * 16, 16)]
            return tuple(
                a + jnp.where(bv >= (g + 1), 1, 0) for g, a in enumerate(accs)
            )

        return lax.fori_loop(0, BSC // 16, binner, accs)

    accs = lax.fori_loop(0, NBCHUNKS, bscan, (zero16,) * 7)
    bounds = [N_NODES - lane_sum(a) for a in accs]

    def chunk(k, cnts):
        base = wid * EPT + k * CE
        pltpu.sync_copy(eflat.at[pl.ds(base, CE)], jv)
        pltpu.sync_copy(eflat.at[pl.ds(N_EDGES + base, CE)], iv)

        def gsub(s, carry):
            o = s * SUB
            pltpu.async_copy(
                pos8.at[jv.at[pl.ds(o, SUB)]], rowsj.at[pl.ds(o, SUB)], sem)
            pltpu.async_copy(
                pos8.at[iv.at[pl.ds(o, SUB)]], rowsi.at[pl.ds(o, SUB)], sem)
            return carry
        lax.fori_loop(0, NSUB, gsub, 0)
        pltpu.make_async_copy(pos8.at[pl.ds(0, CE)], rowsj, sem).wait()
        pltpu.make_async_copy(pos8.at[pl.ds(0, CE)], rowsi, sem).wait()

        def einner(t, cnts):
            ev = t * 16 + lane
            xj = plsc.load_gather(rowsj, [ev, c0])
            yj = plsc.load_gather(rowsj, [ev, c1])
            zj = plsc.load_gather(rowsj, [ev, c2])
            xi = plsc.load_gather(rowsi, [ev, c0])
            yi = plsc.load_gather(rowsi, [ev, c1])
            zi = plsc.load_gather(rowsi, [ev, c2])
            dx = xj - xi
            dy = yj - yi
            dz = zj - zi
            ss = dx * dx + dy * dy + dz * dz + 1e-12
            y = plsc.bitcast(
                0x5F3759DF - (plsc.bitcast(ss, jnp.int32) >> 1), jnp.float32
            )
            h = ss * 0.5
            y = y * (1.5 - h * y * y)
            y = y * (1.5 - h * y * y)
            y = y * (1.5 - h * y * y)
            nrm = ss * y
            plsc.store_scatter(outb, [ev, c0], dx)
            plsc.store_scatter(outb, [ev, c1], dy)
            plsc.store_scatter(outb, [ev, c2], dz)
            plsc.store_scatter(outb, [ev, c3], nrm)
            ivals = iv[pl.ds(t * 16, 16)]
            return tuple(
                cn + jnp.where(ivals >= bounds[g], 1, 0)
                for g, cn in enumerate(cnts)
            )

        cnts = lax.fori_loop(0, CE // 16, einner, cnts)
        pltpu.sync_copy(outb, out.at[pl.ds(base, CE)])
        return cnts

    cnts = lax.fori_loop(0, NCHUNKS, chunk, (zero16,) * 7)

    cge = ([jnp.full((16,), EPT, jnp.int32)]
           + [lane_sum(cn) for cn in cnts]
           + [zero16])
    nb = jnp.zeros((16,), jnp.int32)
    for g in range(N_GRAPHS):
        nb = nb + jnp.where(lane == g, cge[g] - cge[g + 1], 0)
    nbv[...] = nb
    pltpu.sync_copy(nbv, partials.at[wid])


def kernel(pos, edge_index, batch):
    pos8 = jnp.pad(pos, ((0, 0), (0, 5)))
    eflat = edge_index.reshape(2 * N_EDGES)
    out, partials = _sc_body(pos8, eflat, batch)
    neighbors = jnp.sum(partials, axis=0)[:N_GRAPHS].astype(jnp.int32)
    zeros = jnp.zeros((N_EDGES, 3), pos.dtype)
    return (out, zeros, zeros, neighbors)


# speedup vs baseline: 85.3603x; 3.1890x over previous
import functools

import jax
import jax.numpy as jnp
from jax import lax
from jax.experimental import pallas as pl
from jax.experimental.pallas import tpu as pltpu
from jax.experimental.pallas import tpu_sc as plsc

N_NODES = 100000
N_EDGES = 6400000
N_GRAPHS = 8

_INFO = plsc.get_sparse_core_info()
NC = _INFO.num_cores
NS = _INFO.num_subcores
NW = NC * NS
CE = 2048
NBLK = CE // 128
TOTCHUNKS = N_EDGES // CE
MAXCHUNKS = -(-TOTCHUNKS // NW)
SUB = 128
NSUB = CE // SUB
BSC = 20000
NBCHUNKS = N_NODES // BSC

_mesh = plsc.VectorSubcoreMesh(core_axis_name="c", subcore_axis_name="s")


@functools.partial(
    pl.kernel,
    mesh=_mesh,
    compiler_params=pltpu.CompilerParams(
        needs_layout_passes=False, use_tc_tiling_on_sc=False),
    out_type=(
        jax.ShapeDtypeStruct((N_EDGES * 4,), jnp.float32),
        jax.ShapeDtypeStruct((NW, 16), jnp.int32),
    ),
    scratch_types=[
        pltpu.VMEM((CE,), jnp.int32),
        pltpu.VMEM((CE,), jnp.int32),
        pltpu.VMEM((CE, 8), jnp.float32),
        pltpu.VMEM((CE, 8), jnp.float32),
        pltpu.VMEM((CE * 4,), jnp.float32),
        pltpu.VMEM((BSC,), jnp.int32),
        pltpu.VMEM((16,), jnp.int32),
        pltpu.SemaphoreType.DMA,
    ],
)
def _sc_body(pos8, eflat, batch, out, partials,
             jv, iv, rowsj, rowsi, outb, bbuf, nbv, sem):
    cid = lax.axis_index("c")
    sid = lax.axis_index("s")
    wid = sid * NC + cid
    lane = lax.iota(jnp.int32, 16)

    def dyn_gather(v, idx):
        return lax.gather(
            v, idx[:, None],
            dimension_numbers=lax.GatherDimensionNumbers(
                offset_dims=(), collapsed_slice_dims=(0,),
                start_index_map=(0,)),
            slice_sizes=(1,),
            mode=lax.GatherScatterMode.PROMISE_IN_BOUNDS)

    def lane_sum(v):
        for sh in (8, 4, 2, 1):
            v = v + dyn_gather(v, lane ^ sh)
        return v

    zero16 = jnp.zeros((16,), jnp.int32)
    c0 = jnp.full((16,), 0, jnp.int32)
    c1 = jnp.full((16,), 1, jnp.int32)
    c2 = jnp.full((16,), 2, jnp.int32)

    def bscan(b, accs):
        pltpu.sync_copy(batch.at[pl.ds(b * BSC, BSC)], bbuf)

        def binner(t, accs):
            bv = bbuf[pl.ds(t * 16, 16)]
            return tuple(
                a + jnp.where(bv >= (g + 1), 1, 0) for g, a in enumerate(accs)
            )

        return lax.fori_loop(0, BSC // 16, binner, accs)

    accs = lax.fori_loop(0, NBCHUNKS, bscan, (zero16,) * 7)
    bounds = [N_NODES - lane_sum(a) for a in accs]

    def do_chunk(m, cnts):
        base = m * CE
        pltpu.sync_copy(eflat.at[pl.ds(base, CE)], jv)
        pltpu.sync_copy(eflat.at[pl.ds(N_EDGES + base, CE)], iv)

        def gsub(s, carry):
            o = s * SUB
            pltpu.async_copy(
                pos8.at[jv.at[pl.ds(o, SUB)]], rowsj.at[pl.ds(o, SUB)], sem)
            pltpu.async_copy(
                pos8.at[iv.at[pl.ds(o, SUB)]], rowsi.at[pl.ds(o, SUB)], sem)
            return carry
        lax.fori_loop(0, NSUB, gsub, 0)
        pltpu.make_async_copy(pos8.at[pl.ds(0, CE)], rowsj, sem).wait()
        pltpu.make_async_copy(pos8.at[pl.ds(0, CE)], rowsi, sem).wait()

        def einner(t, cnts):
            ev = t * 16 + lane
            xj = plsc.load_gather(rowsj, [ev, c0])
            yj = plsc.load_gather(rowsj, [ev, c1])
            zj = plsc.load_gather(rowsj, [ev, c2])
            xi = plsc.load_gather(rowsi, [ev, c0])
            yi = plsc.load_gather(rowsi, [ev, c1])
            zi = plsc.load_gather(rowsi, [ev, c2])
            dx = xj - xi
            dy = yj - yi
            dz = zj - zi
            ss = dx * dx + dy * dy + dz * dz + 1e-12
            y = plsc.bitcast(
                0x5F3759DF - (plsc.bitcast(ss, jnp.int32) >> 1), jnp.float32
            )
            h = ss * 0.5
            y = y * (1.5 - h * y * y)
            y = y * (1.5 - h * y * y)
            y = y * (1.5 - h * y * y)
            nrm = ss * y
            b = t // 8
            u = t - b * 8
            o0 = b * 512 + u * 16
            outb[pl.ds(o0, 16)] = dx
            outb[pl.ds(o0 + 128, 16)] = dy
            outb[pl.ds(o0 + 256, 16)] = dz
            outb[pl.ds(o0 + 384, 16)] = nrm
            ivals = iv[pl.ds(t * 16, 16)]
            return tuple(
                cn + jnp.where(ivals >= bounds[g], 1, 0)
                for g, cn in enumerate(cnts)
            )

        cnts = lax.fori_loop(0, CE // 16, einner, cnts)
        pltpu.sync_copy(outb, out.at[pl.ds(base * 4, CE * 4)])
        return cnts

    def chunk(k, cnts):
        m = k * NW + wid
        return lax.cond(m < TOTCHUNKS, lambda c: do_chunk(m, c),
                        lambda c: c, cnts)

    cnts = lax.fori_loop(0, MAXCHUNKS, chunk, (zero16,) * 7)

    cge = [lane_sum(cn) for cn in cnts] + [zero16]
    nb = jnp.zeros((16,), jnp.int32)
    for g in range(N_GRAPHS - 1, 0, -1):
        nb = nb + jnp.where(lane == g, cge[g - 1] - cge[g], 0)
    owned_full = (TOTCHUNKS // NW) * CE
    extra = jnp.where(wid < (TOTCHUNKS % NW), CE, 0)
    owned = jnp.full((16,), owned_full, jnp.int32) + extra
    nb = nb + jnp.where(lane == 0, owned - cge[0], 0)
    nbv[...] = nb
    pltpu.sync_copy(nbv, partials.at[wid])


def kernel(pos, edge_index, batch):
    pos8 = jnp.pad(pos, ((0, 0), (0, 5)))
    eflat = edge_index.reshape(2 * N_EDGES)
    flat, partials = _sc_body(pos8, eflat, batch)
    out = flat.reshape(N_EDGES // 128, 4, 128).transpose(0, 2, 1).reshape(
        N_EDGES, 4)
    neighbors = jnp.sum(partials, axis=0)[:N_GRAPHS].astype(jnp.int32)
    zeros = jnp.zeros((N_EDGES, 3), pos.dtype)
    return (out, zeros, zeros, neighbors)


# speedup vs baseline: 139.5235x; 1.6345x over previous
import functools

import jax
import jax.numpy as jnp
from jax import lax
from jax.experimental import pallas as pl
from jax.experimental.pallas import tpu as pltpu
from jax.experimental.pallas import tpu_sc as plsc

N_NODES = 100000
N_EDGES = 6400000
N_GRAPHS = 8

_INFO = plsc.get_sparse_core_info()
NC = _INFO.num_cores
NS = _INFO.num_subcores
NW = NC * NS
CE = 2048
NBLK = CE // 128
TOTCHUNKS = N_EDGES // CE
MAXCHUNKS = -(-TOTCHUNKS // NW)
SUB = 128
NSUB = CE // SUB
BSC = 20000
NBCHUNKS = N_NODES // BSC

_mesh = plsc.VectorSubcoreMesh(core_axis_name="c", subcore_axis_name="s")


@functools.partial(
    pl.kernel,
    mesh=_mesh,
    compiler_params=pltpu.CompilerParams(
        needs_layout_passes=False, use_tc_tiling_on_sc=False),
    out_type=(
        jax.ShapeDtypeStruct((N_EDGES * 4,), jnp.float32),
        jax.ShapeDtypeStruct((NW, 16), jnp.int32),
    ),
    scratch_types=[
        pltpu.VMEM((CE,), jnp.int32),
        pltpu.VMEM((CE,), jnp.int32),
        pltpu.VMEM((CE,), jnp.int32),
        pltpu.VMEM((CE,), jnp.int32),
        pltpu.VMEM((CE, 8), jnp.float32),
        pltpu.VMEM((CE, 8), jnp.float32),
        pltpu.VMEM((CE, 8), jnp.float32),
        pltpu.VMEM((CE, 8), jnp.float32),
        pltpu.VMEM((CE * 4,), jnp.float32),
        pltpu.VMEM((BSC,), jnp.int32),
        pltpu.VMEM((16,), jnp.int32),
        pltpu.SemaphoreType.DMA,
        pltpu.SemaphoreType.DMA,
    ],
)
def _sc_body(pos8, eflat, batch, out, partials,
             jv0, iv0, jv1, iv1, rowsj0, rowsi0, rowsj1, rowsi1,
             outb, bbuf, nbv, sem0, sem1):
    cid = lax.axis_index("c")
    sid = lax.axis_index("s")
    wid = sid * NC + cid
    lane = lax.iota(jnp.int32, 16)

    def dyn_gather(v, idx):
        return lax.gather(
            v, idx[:, None],
            dimension_numbers=lax.GatherDimensionNumbers(
                offset_dims=(), collapsed_slice_dims=(0,),
                start_index_map=(0,)),
            slice_sizes=(1,),
            mode=lax.GatherScatterMode.PROMISE_IN_BOUNDS)

    def lane_sum(v):
        for sh in (8, 4, 2, 1):
            v = v + dyn_gather(v, lane ^ sh)
        return v

    zero16 = jnp.zeros((16,), jnp.int32)
    c0 = jnp.full((16,), 0, jnp.int32)
    c1 = jnp.full((16,), 1, jnp.int32)
    c2 = jnp.full((16,), 2, jnp.int32)

    def bscan(b, accs):
        pltpu.sync_copy(batch.at[pl.ds(b * BSC, BSC)], bbuf)

        def binner(t, accs):
            bv = bbuf[pl.ds(t * 16, 16)]
            return tuple(
                a + jnp.where(bv >= (g + 1), 1, 0) for g, a in enumerate(accs)
            )

        return lax.fori_loop(0, BSC // 16, binner, accs)

    accs = lax.fori_loop(0, NBCHUNKS, bscan, (zero16,) * 7)
    bounds = [N_NODES - lane_sum(a) for a in accs]

    def fire_chunk(m, jvb, ivb, rjb, rib, semg):
        base = m * CE
        pltpu.sync_copy(eflat.at[pl.ds(base, CE)], jvb)
        pltpu.sync_copy(eflat.at[pl.ds(N_EDGES + base, CE)], ivb)

        def gsub(s, carry):
            o = s * SUB
            pltpu.async_copy(
                pos8.at[jvb.at[pl.ds(o, SUB)]], rjb.at[pl.ds(o, SUB)], semg)
            pltpu.async_copy(
                pos8.at[ivb.at[pl.ds(o, SUB)]], rib.at[pl.ds(o, SUB)], semg)
            return carry
        lax.fori_loop(0, NSUB, gsub, 0)

    def proc_chunk(m, ivb, rjb, rib, semg, cnts):
        base = m * CE
        pltpu.make_async_copy(pos8.at[pl.ds(0, CE)], rjb, semg).wait()
        pltpu.make_async_copy(pos8.at[pl.ds(0, CE)], rib, semg).wait()

        def einner(t, cnts):
            ev = t * 16 + lane
            xj = plsc.load_gather(rjb, [ev, c0])
            yj = plsc.load_gather(rjb, [ev, c1])
            zj = plsc.load_gather(rjb, [ev, c2])
            xi = plsc.load_gather(rib, [ev, c0])
            yi = plsc.load_gather(rib, [ev, c1])
            zi = plsc.load_gather(rib, [ev, c2])
            dx = xj - xi
            dy = yj - yi
            dz = zj - zi
            ss = dx * dx + dy * dy + dz * dz + 1e-12
            y = plsc.bitcast(
                0x5F3759DF - (plsc.bitcast(ss, jnp.int32) >> 1), jnp.float32
            )
            h = ss * 0.5
            y = y * (1.5 - h * y * y)
            y = y * (1.5 - h * y * y)
            y = y * (1.5 - h * y * y)
            nrm = ss * y
            b = t // 8
            u = t - b * 8
            o0 = b * 512 + u * 16
            outb[pl.ds(o0, 16)] = dx
            outb[pl.ds(o0 + 128, 16)] = dy
            outb[pl.ds(o0 + 256, 16)] = dz
            outb[pl.ds(o0 + 384, 16)] = nrm
            ivals = ivb[pl.ds(t * 16, 16)]
            return tuple(
                cn + jnp.where(ivals >= bounds[g], 1, 0)
                for g, cn in enumerate(cnts)
            )

        cnts = lax.fori_loop(0, CE // 16, einner, cnts)
        pltpu.sync_copy(outb, out.at[pl.ds(base * 4, CE * 4)])
        return cnts

    fire_chunk(wid, jv0, iv0, rowsj0, rowsi0, sem0)

    def pair(p, cnts):
        m0 = (2 * p) * NW + wid
        m1 = m0 + NW
        m2 = m1 + NW

        @pl.when(m1 < TOTCHUNKS)
        def _():
            fire_chunk(m1, jv1, iv1, rowsj1, rowsi1, sem1)

        cnts = lax.cond(
            m0 < TOTCHUNKS,
            lambda c: proc_chunk(m0, iv0, rowsj0, rowsi0, sem0, c),
            lambda c: c, cnts)

        @pl.when(m2 < TOTCHUNKS)
        def _():
            fire_chunk(m2, jv0, iv0, rowsj0, rowsi0, sem0)

        cnts = lax.cond(
            m1 < TOTCHUNKS,
            lambda c: proc_chunk(m1, iv1, rowsj1, rowsi1, sem1, c),
            lambda c: c, cnts)
        return cnts

    cnts = lax.fori_loop(0, MAXCHUNKS // 2, pair, (zero16,) * 7)

    cge = [lane_sum(cn) for cn in cnts] + [zero16]
    nb = jnp.zeros((16,), jnp.int32)
    for g in range(N_GRAPHS - 1, 0, -1):
        nb = nb + jnp.where(lane == g, cge[g - 1] - cge[g], 0)
    owned_full = (TOTCHUNKS // NW) * CE
    extra = jnp.where(wid < (TOTCHUNKS % NW), CE, 0)
    owned = jnp.full((16,), owned_full, jnp.int32) + extra
    nb = nb + jnp.where(lane == 0, owned - cge[0], 0)
    nbv[...] = nb
    pltpu.sync_copy(nbv, partials.at[wid])


def kernel(pos, edge_index, batch):
    pos8 = jnp.pad(pos, ((0, 0), (0, 5)))
    eflat = edge_index.reshape(2 * N_EDGES)
    flat, partials = _sc_body(pos8, eflat, batch)
    out = flat.reshape(N_EDGES // 128, 4, 128).transpose(0, 2, 1).reshape(
        N_EDGES, 4)
    neighbors = jnp.sum(partials, axis=0)[:N_GRAPHS].astype(jnp.int32)
    zeros = jnp.zeros((N_EDGES, 3), pos.dtype)
    return (out, zeros, zeros, neighbors)


# speedup vs baseline: 143.6018x; 1.0292x over previous
import functools

import jax
import jax.numpy as jnp
from jax import lax
from jax.experimental import pallas as pl
from jax.experimental.pallas import tpu as pltpu
from jax.experimental.pallas import tpu_sc as plsc

N_NODES = 100000
N_EDGES = 6400000
N_GRAPHS = 8

_INFO = plsc.get_sparse_core_info()
NC = _INFO.num_cores
NS = _INFO.num_subcores
NW = NC * NS
CE = 2048
NBLK = CE // 128
TOTCHUNKS = N_EDGES // CE
MAXCHUNKS = -(-TOTCHUNKS // NW)
SUB = 128
NSUB = CE // SUB
BSC = 20000
NBCHUNKS = N_NODES // BSC

_mesh = plsc.VectorSubcoreMesh(core_axis_name="c", subcore_axis_name="s")


@functools.partial(
    pl.kernel,
    mesh=_mesh,
    compiler_params=pltpu.CompilerParams(
        needs_layout_passes=False, use_tc_tiling_on_sc=False),
    out_type=(
        jax.ShapeDtypeStruct((N_EDGES * 4,), jnp.float32),
        jax.ShapeDtypeStruct((NW, 16), jnp.int32),
    ),
    scratch_types=[
        pltpu.VMEM((CE,), jnp.int32),
        pltpu.VMEM((CE,), jnp.int32),
        pltpu.VMEM((CE,), jnp.int32),
        pltpu.VMEM((CE,), jnp.int32),
        pltpu.VMEM((CE, 8), jnp.float32),
        pltpu.VMEM((CE, 8), jnp.float32),
        pltpu.VMEM((CE, 8), jnp.float32),
        pltpu.VMEM((CE, 8), jnp.float32),
        pltpu.VMEM((CE * 4,), jnp.float32),
        pltpu.VMEM((CE * 4,), jnp.float32),
        pltpu.VMEM((BSC,), jnp.int32),
        pltpu.VMEM((16,), jnp.int32),
        pltpu.SemaphoreType.DMA,
        pltpu.SemaphoreType.DMA,
        pltpu.SemaphoreType.DMA,
        pltpu.SemaphoreType.DMA,
    ],
)
def _sc_body(pos8, eflat, batch, out, partials,
             jv0, iv0, jv1, iv1, rowsj0, rowsi0, rowsj1, rowsi1,
             outb0, outb1, bbuf, nbv, sem0, sem1, semw0, semw1):
    cid = lax.axis_index("c")
    sid = lax.axis_index("s")
    wid = sid * NC + cid
    lane = lax.iota(jnp.int32, 16)

    def dyn_gather(v, idx):
        return lax.gather(
            v, idx[:, None],
            dimension_numbers=lax.GatherDimensionNumbers(
                offset_dims=(), collapsed_slice_dims=(0,),
                start_index_map=(0,)),
            slice_sizes=(1,),
            mode=lax.GatherScatterMode.PROMISE_IN_BOUNDS)

    def lane_sum(v):
        for sh in (8, 4, 2, 1):
            v = v + dyn_gather(v, lane ^ sh)
        return v

    zero16 = jnp.zeros((16,), jnp.int32)
    c0 = jnp.full((16,), 0, jnp.int32)
    c1 = jnp.full((16,), 1, jnp.int32)
    c2 = jnp.full((16,), 2, jnp.int32)

    def bscan(b, accs):
        pltpu.sync_copy(batch.at[pl.ds(b * BSC, BSC)], bbuf)

        def binner(t, accs):
            bv = bbuf[pl.ds(t * 16, 16)]
            return tuple(
                a + jnp.where(bv >= (g + 1), 1, 0) for g, a in enumerate(accs)
            )

        return lax.fori_loop(0, BSC // 16, binner, accs)

    accs = lax.fori_loop(0, NBCHUNKS, bscan, (zero16,) * 7)
    bounds = [N_NODES - lane_sum(a) for a in accs]

    def fire_chunk(m, jvb, ivb, rjb, rib, semg):
        base = m * CE
        pltpu.sync_copy(eflat.at[pl.ds(base, CE)], jvb)
        pltpu.sync_copy(eflat.at[pl.ds(N_EDGES + base, CE)], ivb)

        def gsub(s, carry):
            o = s * SUB
            pltpu.async_copy(
                pos8.at[jvb.at[pl.ds(o, SUB)]], rjb.at[pl.ds(o, SUB)], semg)
            pltpu.async_copy(
                pos8.at[ivb.at[pl.ds(o, SUB)]], rib.at[pl.ds(o, SUB)], semg)
            return carry
        lax.fori_loop(0, NSUB, gsub, 0)

    def proc_chunk(m, ivb, rjb, rib, outbb, semg, semw, first, cnts):
        base = m * CE
        pltpu.make_async_copy(pos8.at[pl.ds(0, CE)], rjb, semg).wait()
        pltpu.make_async_copy(pos8.at[pl.ds(0, CE)], rib, semg).wait()

        @pl.when(jnp.logical_not(first))
        def _():
            pltpu.make_async_copy(
                outbb, out.at[pl.ds(0, CE * 4)], semw).wait()

        def einner(t, cnts):
            ev = t * 16 + lane
            xj = plsc.load_gather(rjb, [ev, c0])
            yj = plsc.load_gather(rjb, [ev, c1])
            zj = plsc.load_gather(rjb, [ev, c2])
            xi = plsc.load_gather(rib, [ev, c0])
            yi = plsc.load_gather(rib, [ev, c1])
            zi = plsc.load_gather(rib, [ev, c2])
            dx = xj - xi
            dy = yj - yi
            dz = zj - zi
            ss = dx * dx + dy * dy + dz * dz + 1e-12
            y = plsc.bitcast(
                0x5F3759DF - (plsc.bitcast(ss, jnp.int32) >> 1), jnp.float32
            )
            h = ss * 0.5
            y = y * (1.5 - h * y * y)
            y = y * (1.5 - h * y * y)
            y = y * (1.5 - h * y * y)
            nrm = ss * y
            b = t // 8
            u = t - b * 8
            o0 = b * 512 + u * 16
            outbb[pl.ds(o0, 16)] = dx
            outbb[pl.ds(o0 + 128, 16)] = dy
            outbb[pl.ds(o0 + 256, 16)] = dz
            outbb[pl.ds(o0 + 384, 16)] = nrm
            ivals = ivb[pl.ds(t * 16, 16)]
            return tuple(
                cn + jnp.where(ivals >= bounds[g], 1, 0)
                for g, cn in enumerate(cnts)
            )

        cnts = lax.fori_loop(0, CE // 16, einner, cnts)
        pltpu.async_copy(outbb, out.at[pl.ds(base * 4, CE * 4)], semw)
        return cnts

    fire_chunk(wid, jv0, iv0, rowsj0, rowsi0, sem0)

    def pair(p, cnts):
        m0 = (2 * p) * NW + wid
        m1 = m0 + NW
        m2 = m1 + NW

        @pl.when(m1 < TOTCHUNKS)
        def _():
            fire_chunk(m1, jv1, iv1, rowsj1, rowsi1, sem1)

        cnts = lax.cond(
            m0 < TOTCHUNKS,
            lambda c: proc_chunk(m0, iv0, rowsj0, rowsi0, outb0, sem0,
                                 semw0, m0 < NW, c),
            lambda c: c, cnts)

        @pl.when(m2 < TOTCHUNKS)
        def _():
            fire_chunk(m2, jv0, iv0, rowsj0, rowsi0, sem0)

        cnts = lax.cond(
            m1 < TOTCHUNKS,
            lambda c: proc_chunk(m1, iv1, rowsj1, rowsi1, outb1, sem1,
                                 semw1, m1 < 2 * NW, c),
            lambda c: c, cnts)
        return cnts

    cnts = lax.fori_loop(0, MAXCHUNKS // 2, pair, (zero16,) * 7)

    pltpu.make_async_copy(outb0, out.at[pl.ds(0, CE * 4)], semw0).wait()
    pltpu.make_async_copy(outb1, out.at[pl.ds(0, CE * 4)], semw1).wait()

    cge = [lane_sum(cn) for cn in cnts] + [zero16]
    nb = jnp.zeros((16,), jnp.int32)
    for g in range(N_GRAPHS - 1, 0, -1):
        nb = nb + jnp.where(lane == g, cge[g - 1] - cge[g], 0)
    owned_full = (TOTCHUNKS // NW) * CE
    extra = jnp.where(wid < (TOTCHUNKS % NW), CE, 0)
    owned = jnp.full((16,), owned_full, jnp.int32) + extra
    nb = nb + jnp.where(lane == 0, owned - cge[0], 0)
    nbv[...] = nb
    pltpu.sync_copy(nbv, partials.at[wid])


def kernel(pos, edge_index, batch):
    pos8 = jnp.pad(pos, ((0, 0), (0, 5)))
    eflat = edge_index.reshape(2 * N_EDGES)
    flat, partials = _sc_body(pos8, eflat, batch)
    out = flat.reshape(N_EDGES // 128, 4, 128).transpose(0, 2, 1).reshape(
        N_EDGES, 4)
    neighbors = jnp.sum(partials, axis=0)[:N_GRAPHS].astype(jnp.int32)
    zeros = jnp.zeros((N_EDGES, 3), pos.dtype)
    return (out, zeros, zeros, neighbors)
